# baseline (speedup 1.0000x reference)
import functools

import jax
import jax.numpy as jnp
from jax import lax
from jax.experimental import pallas as pl
from jax.experimental.pallas import tpu as pltpu
from jax.experimental.pallas import tpu_sc as plsc

N = 10000
E = 320000
D = 128

NTILES = 32
ROWS = 320
NPAD = NTILES * ROWS // 1 * 1
EB = 640
NBLK = E // EB
G = 16

EA = E // NTILES
NPADP = NPAD + 16
SEG = NPAD // 16

_mesh = plsc.VectorSubcoreMesh(core_axis_name="c", subcore_axis_name="s")


def _deg_body(dst_hbm, out_deg, dst_v, acc_v, tmp_v, seg_v, hist_sh):
    c = lax.axis_index("c")
    s = lax.axis_index("s")
    wid = s * 2 + c
    zero16f = jnp.zeros((16,), jnp.float32)

    def z(i, carry):
        acc_v[pl.ds(i * 16, 16)] = zero16f
        return carry

    lax.fori_loop(0, NPADP // 16, z, 0)
    pltpu.sync_copy(dst_hbm.at[pl.ds(wid * EA, EA)], dst_v)
    e0 = (lax.iota(jnp.int32, 16) == 0).astype(jnp.float32)

    def per16(i, carry):
        dv = dst_v[pl.ds(i * 16, 16)]
        for k in range(16):
            d = dv[k]
            acc_v[pl.ds(d, 16)] = acc_v[pl.ds(d, 16)] + e0
        return carry

    lax.fori_loop(0, EA // 16, per16, 0)
    pltpu.sync_copy(acc_v, hist_sh.at[s])
    plsc.subcore_barrier()

    def zs(i, carry):
        seg_v[pl.ds(i * 16, 16)] = zero16f
        return carry

    lax.fori_loop(0, SEG // 16, zs, 0)
    for t in range(16):
        pltpu.sync_copy(hist_sh.at[t, pl.ds(s * SEG, SEG)], tmp_v)

        def add1(i, carry):
            seg_v[pl.ds(i * 16, 16)] = (seg_v[pl.ds(i * 16, 16)]
                                        + tmp_v[pl.ds(i * 16, 16)])
            return carry

        lax.fori_loop(0, SEG // 16, add1, 0)
    pltpu.sync_copy(seg_v, out_deg.at[pl.ds(c * NPAD + s * SEG, SEG)])


def _deg_stage(dst):
    k = pl.kernel(
        _deg_body,
        mesh=_mesh,
        compiler_params=pltpu.CompilerParams(needs_layout_passes=False),
        out_type=jax.ShapeDtypeStruct((2 * NPAD,), jnp.float32),
        scratch_types=[
            pltpu.VMEM((EA,), jnp.int32),
            pltpu.VMEM((NPADP,), jnp.float32),
            pltpu.VMEM((SEG,), jnp.float32),
            pltpu.VMEM((SEG,), jnp.float32),
            pltpu.VMEM_SHARED((16, NPADP), jnp.float32),
        ],
    )
    return k(dst)


def _seg_body(src_hbm, dst_hbm, xz_hbm, out_sum, out_max,
              src_v, dst_v, msrc_v, mdl_v, rows_v, accs_v, accm_v, sem, sem_s):
    c = lax.axis_index("c")
    s = lax.axis_index("s")
    wid = s * 2 + c
    lo = wid * ROWS

    zero16f = jnp.zeros((16,), jnp.float32)
    ninf16 = jnp.full((16,), -jnp.inf, jnp.float32)

    def z1(i, carry):
        accs_v[pl.ds(i * 16, 16)] = zero16f
        return carry

    lax.fori_loop(0, ROWS * 256 // 16, z1, 0)

    def z2(i, carry):
        accm_v[pl.ds(i * 16, 16)] = ninf16
        return carry

    lax.fori_loop(0, ROWS * 128 // 16, z2, 0)

    pltpu.async_copy(src_hbm.at[pl.ds(0, EB)], src_v.at[0], sem_s)
    pltpu.async_copy(dst_hbm.at[pl.ds(0, EB)], dst_v.at[0], sem_s)

    def blk(b, carry):
        cur = b & 1
        pltpu.make_async_copy(src_hbm.at[pl.ds(b * EB, EB)],
                              src_v.at[cur], sem_s).wait()
        pltpu.make_async_copy(dst_hbm.at[pl.ds(b * EB, EB)],
                              dst_v.at[cur], sem_s).wait()

        @pl.when(b + 1 < NBLK)
        def _():
            nxt = 1 - cur
            pltpu.async_copy(src_hbm.at[pl.ds((b + 1) * EB, EB)],
                             src_v.at[nxt], sem_s)
            pltpu.async_copy(dst_hbm.at[pl.ds((b + 1) * EB, EB)],
                             dst_v.at[nxt], sem_s)

        def filt(i, off):
            sv = src_v[cur, pl.ds(i * 16, 16)]
            dv = dst_v[cur, pl.ds(i * 16, 16)]
            msk = (dv >= lo) & (dv < lo + ROWS)
            pk = jnp.where(msk, sv * 1024 + (dv - lo), 0)
            key = jnp.where(msk, 0, 1)
            _, pks = plsc.sort_key_val(key, pk)
            msrc_v[pl.ds(off, 16)] = pks >> 10
            mdl_v[pl.ds(off, 16)] = pks & 1023
            return off + jnp.sum(msk.astype(jnp.int32))

        m = lax.fori_loop(0, EB // 16, filt, jnp.int32(0))
        nch = (m + G - 1) // G

        def chunk(cc, carry2):
            idxv = msrc_v[pl.ds(cc * G, 16)]
            pltpu.async_copy(xz_hbm.at[idxv], rows_v, sem).wait()
            nin = jnp.minimum(m - cc * G, G)

            def one(jj, carry3):
                dl = mdl_v[pl.ds(cc * G + jj, 16)][0]
                for k in range(8):
                    r = rows_v[jj, pl.ds(k * 16, 16)]
                    plsc.addupdate(accs_v.at[pl.ds(dl * 256 + k * 16, 16)], r)
                    mo = accm_v[pl.ds(dl * 128 + k * 16, 16)]
                    accm_v[pl.ds(dl * 128 + k * 16, 16)] = jnp.maximum(mo, r)
                for k in range(8, 16):
                    r = rows_v[jj, pl.ds(k * 16, 16)]
                    plsc.addupdate(accs_v.at[pl.ds(dl * 256 + k * 16, 16)], r)
                return carry3

            lax.fori_loop(0, nin, one, 0)
            return carry2

        lax.fori_loop(0, nch, chunk, 0)
        return carry

    lax.fori_loop(0, NBLK, blk, 0)

    pltpu.sync_copy(accs_v, out_sum.at[pl.ds(lo * 256, ROWS * 256)])
    pltpu.sync_copy(accm_v, out_max.at[pl.ds(lo * 128, ROWS * 128)])


def _seg_stage(src, dst, xz):
    k = pl.kernel(
        _seg_body,
        mesh=_mesh,
        compiler_params=pltpu.CompilerParams(needs_layout_passes=False),
        out_type=(
            jax.ShapeDtypeStruct((NPAD * 256,), jnp.float32),
            jax.ShapeDtypeStruct((NPAD * 128,), jnp.float32),
        ),
        scratch_types=[
            pltpu.VMEM((2, EB), jnp.int32),
            pltpu.VMEM((2, EB), jnp.int32),
            pltpu.VMEM((EB + 16,), jnp.int32),
            pltpu.VMEM((EB + 16,), jnp.int32),
            pltpu.VMEM((G, 256), jnp.float32),
            pltpu.VMEM((ROWS * 256,), jnp.float32),
            pltpu.VMEM((ROWS * 128,), jnp.float32),
            pltpu.SemaphoreType.DMA,
            pltpu.SemaphoreType.DMA,
        ],
    )
    return k(src, dst, xz)


def _prep_body(dega_ref, degb_ref, x_ref, xz_ref, dinv_ref, smean_ref):
    deg = dega_ref[...] + degb_ref[...]
    dinv = lax.rsqrt(deg + 1.0)
    xv = x_ref[...]
    xz_ref[:, :D] = xv
    xz_ref[:, D:] = xv * dinv
    dinv_ref[...] = dinv
    smean_ref[...] = 1.0 / jnp.maximum(deg, 1.0)


def _prep_stage(dega, degb, xpad):
    B = 1024
    grid = (NPAD // B,)
    col1 = pl.BlockSpec((B, 1), lambda i: (i, 0))
    return pl.pallas_call(
        _prep_body,
        grid=grid,
        in_specs=[col1, col1, pl.BlockSpec((B, D), lambda i: (i, 0))],
        out_specs=[pl.BlockSpec((B, 2 * D), lambda i: (i, 0)), col1, col1],
        out_shape=[
            jax.ShapeDtypeStruct((NPAD, 2 * D), jnp.float32),
            jax.ShapeDtypeStruct((NPAD, 1), jnp.float32),
            jax.ShapeDtypeStruct((NPAD, 1), jnp.float32),
        ],
    )(dega, degb, xpad)


def _elu(h):
    return jnp.where(h > 0, h, jnp.exp(jnp.minimum(h, 0.0)) - 1.0)


def _dense_body(wmix_ref, x_ref, s_ref, mx_ref, dinv_ref, smean_ref,
                Wmlp, bmlp, Wsx, Wsm, bs, Wtx, Wtm, bt, Wux, Wum, bu,
                Wg, bg, Wg1, bg1, Wg2, bg2, out_ref):
    x = x_ref[...]
    msum = s_ref[:, :D]
    gsum = s_ref[:, D:]
    dinv = dinv_ref[...]
    mean = msum * smean_ref[...]
    mxv = mx_ref[...]
    mxv = jnp.where(mxv > -1e30, mxv, 0.0)
    agg = gsum * dinv + x * (dinv * dinv)
    f32 = jnp.float32
    h0 = jnp.dot(x, Wmlp[...], preferred_element_type=f32) + bmlp[...]
    h1 = (jnp.dot(x, Wsx[...], preferred_element_type=f32)
          + jnp.dot(mean, Wsm[...], preferred_element_type=f32) + bs[...])
    h2 = (jnp.dot(x, Wtx[...], preferred_element_type=f32)
          + jnp.dot(msum, Wtm[...], preferred_element_type=f32) + bt[...])
    h3 = (jnp.dot(x, Wux[...], preferred_element_type=f32)
          + jnp.dot(mxv, Wum[...], preferred_element_type=f32) + bu[...])
    h4 = jnp.dot(agg, Wg[...], preferred_element_type=f32) + bg[...]
    g = jnp.maximum(jnp.dot(x + msum, Wg1[...], preferred_element_type=f32) + bg1[...], 0.0)
    h5 = jnp.dot(g, Wg2[...], preferred_element_type=f32) + bg2[...]
    w = wmix_ref[...]
    out_ref[...] = (w[0, 0] * _elu(h0) + w[0, 1] * _elu(h1) + w[0, 2] * _elu(h2)
                    + w[0, 3] * _elu(h3) + w[0, 4] * _elu(h4) + w[0, 5] * _elu(h5))


def _dense_stage(wmix, x, S, MX, dinv, smean, *weights):
    B = 1000
    grid = (N // B,)
    row = pl.BlockSpec((B, D), lambda i: (i, 0))
    row2 = pl.BlockSpec((B, 2 * D), lambda i: (i, 0))
    col1 = pl.BlockSpec((B, 1), lambda i: (i, 0))
    full = lambda a: pl.BlockSpec(a.shape, lambda i: (0,) * a.ndim)
    wspecs = [full(w) for w in weights]
    return pl.pallas_call(
        _dense_body,
        grid=grid,
        in_specs=[full(wmix), row, row2, row, col1, col1] + wspecs,
        out_specs=row,
        out_shape=jax.ShapeDtypeStruct((N, D), jnp.float32),
    )(wmix, x, S, MX, dinv, smean, *weights)


def kernel(x, weights, edge_index, W_mlp, b_mlp, W_sage, b_sage, W_ssum, b_ssum,
           W_smax, b_smax, W_gcn, b_gcn, W_gin1, b_gin1, W_gin2, b_gin2):
    src = edge_index[0]
    dst = edge_index[1]

    deg_flat = _deg_stage(dst)
    dega = deg_flat[:NPAD].reshape(NPAD, 1)
    degb = deg_flat[NPAD:].reshape(NPAD, 1)

    xpad = jnp.pad(x, ((0, NPAD - N), (0, 0)))
    xz, dinv, smean = _prep_stage(dega, degb, xpad)

    sums_flat, mx_flat = _seg_stage(src, dst, xz)
    S = sums_flat.reshape(NPAD, 2 * D)
    MX = mx_flat.reshape(NPAD, D)

    wmix = jnp.zeros((1, 128), jnp.float32).at[0, :6].set(weights)
    b = lambda v: v.reshape(1, D)
    return _dense_stage(wmix, x, S, MX, dinv, smean,
                        W_mlp, b(b_mlp),
                        W_sage[:D], W_sage[D:], b(b_sage),
                        W_ssum[:D], W_ssum[D:], b(b_ssum),
                        W_smax[:D], W_smax[D:], b(b_smax),
                        W_gcn, b(b_gcn),
                        W_gin1, b(b_gin1),
                        W_gin2, b(b_gin2))


# speedup vs baseline: 1.7321x; 1.7321x over previous
import functools

import jax
import jax.numpy as jnp
from jax import lax
from jax.experimental import pallas as pl
from jax.experimental.pallas import tpu as pltpu
from jax.experimental.pallas import tpu_sc as plsc

N = 10000
E = 320000
D = 128

NTILES = 32
ROWS = 320
NPAD = NTILES * ROWS // 1 * 1
EB = 1000
NBLK = E // EB
G = 16

EA = E // NTILES
NPADP = NPAD + 16
SEG = NPAD // 16

_mesh = plsc.VectorSubcoreMesh(core_axis_name="c", subcore_axis_name="s")


def _deg_body(dst_hbm, out_deg, dst_v, acc_v, tmp_v, seg_v, hist_sh):
    c = lax.axis_index("c")
    s = lax.axis_index("s")
    wid = s * 2 + c
    zero16f = jnp.zeros((16,), jnp.float32)

    def z(i, carry):
        acc_v[pl.ds(i * 16, 16)] = zero16f
        return carry

    lax.fori_loop(0, NPADP // 16, z, 0)
    pltpu.sync_copy(dst_hbm.at[pl.ds(wid * EA, EA)], dst_v)
    e0 = (lax.iota(jnp.int32, 16) == 0).astype(jnp.float32)

    def per16(i, carry):
        dv = dst_v[pl.ds(i * 16, 16)]
        for k in range(16):
            d = dv[k]
            acc_v[pl.ds(d, 16)] = acc_v[pl.ds(d, 16)] + e0
        return carry

    lax.fori_loop(0, EA // 16, per16, 0)
    pltpu.sync_copy(acc_v, hist_sh.at[s])
    plsc.subcore_barrier()

    def zs(i, carry):
        seg_v[pl.ds(i * 16, 16)] = zero16f
        return carry

    lax.fori_loop(0, SEG // 16, zs, 0)
    for t in range(16):
        pltpu.sync_copy(hist_sh.at[t, pl.ds(s * SEG, SEG)], tmp_v)

        def add1(i, carry):
            seg_v[pl.ds(i * 16, 16)] = (seg_v[pl.ds(i * 16, 16)]
                                        + tmp_v[pl.ds(i * 16, 16)])
            return carry

        lax.fori_loop(0, SEG // 16, add1, 0)
    pltpu.sync_copy(seg_v, out_deg.at[pl.ds(c * NPAD + s * SEG, SEG)])


def _deg_stage(dst):
    k = pl.kernel(
        _deg_body,
        mesh=_mesh,
        compiler_params=pltpu.CompilerParams(needs_layout_passes=False),
        out_type=jax.ShapeDtypeStruct((2 * NPAD,), jnp.float32),
        scratch_types=[
            pltpu.VMEM((EA,), jnp.int32),
            pltpu.VMEM((NPADP,), jnp.float32),
            pltpu.VMEM((SEG,), jnp.float32),
            pltpu.VMEM((SEG,), jnp.float32),
            pltpu.VMEM_SHARED((16, NPADP), jnp.float32),
        ],
    )
    return k(dst)


def _seg_body(ei_hbm, xz_hbm, out_sum, out_max,
              ei_v, msrc_v, mdl_v, rows_v, accs_v, accm_v, sem):
    c = lax.axis_index("c")
    s = lax.axis_index("s")
    wid = s * 2 + c
    lo = wid * ROWS

    zero16f = jnp.zeros((16,), jnp.float32)
    ninf16 = jnp.full((16,), -jnp.inf, jnp.float32)

    def z1(i, carry):
        accs_v[pl.ds(i * 16, 16)] = zero16f
        return carry

    lax.fori_loop(0, ROWS * 256 // 16, z1, 0)

    def z2(i, carry):
        accm_v[pl.ds(i * 16, 16)] = ninf16
        return carry

    lax.fori_loop(0, ROWS * 128 // 16, z2, 0)

    def blk(b, carry):
        pltpu.sync_copy(ei_hbm.at[b], ei_v)

        def filt(i, off):
            sv = ei_v[0, pl.ds(i * 16, 16)]
            dv = ei_v[1, pl.ds(i * 16, 16)]
            msk = (dv >= lo) & (dv < lo + ROWS)
            pk = jnp.where(msk, sv * 1024 + (dv - lo), 0)
            key = jnp.where(msk, 0, 1)
            _, pks = plsc.sort_key_val(key, pk)
            msrc_v[pl.ds(off, 16)] = pks >> 10
            mdl_v[pl.ds(off, 16)] = pks & 1023
            return off + jnp.sum(msk.astype(jnp.int32))

        m = lax.fori_loop(0, EB // 16, filt, jnp.int32(0))
        nch = (m + G - 1) // G

        def chunk(cc, carry2):
            idxv = msrc_v[pl.ds(cc * G, 16)]
            pltpu.async_copy(xz_hbm.at[idxv], rows_v, sem).wait()
            nin = jnp.minimum(m - cc * G, G)

            def one(jj, carry3):
                dl = mdl_v[pl.ds(cc * G + jj, 16)][0]
                for k in range(8):
                    r = rows_v[jj, pl.ds(k * 16, 16)]
                    plsc.addupdate(accs_v.at[pl.ds(dl * 256 + k * 16, 16)], r)
                    mo = accm_v[pl.ds(dl * 128 + k * 16, 16)]
                    accm_v[pl.ds(dl * 128 + k * 16, 16)] = jnp.maximum(mo, r)
                for k in range(8, 16):
                    r = rows_v[jj, pl.ds(k * 16, 16)]
                    plsc.addupdate(accs_v.at[pl.ds(dl * 256 + k * 16, 16)], r)
                return carry3

            lax.fori_loop(0, nin, one, 0)
            return carry2

        lax.fori_loop(0, nch, chunk, 0)
        return carry

    lax.fori_loop(0, NBLK, blk, 0)

    pltpu.sync_copy(accs_v, out_sum.at[pl.ds(lo * 256, ROWS * 256)])
    pltpu.sync_copy(accm_v, out_max.at[pl.ds(lo * 128, ROWS * 128)])


def _seg_stage(ei, xz):
    k = pl.kernel(
        _seg_body,
        mesh=_mesh,
        compiler_params=pltpu.CompilerParams(needs_layout_passes=False),
        out_type=(
            jax.ShapeDtypeStruct((NPAD * 256,), jnp.float32),
            jax.ShapeDtypeStruct((NPAD * 128,), jnp.float32),
        ),
        scratch_types=[
            pltpu.VMEM((2, EB), jnp.int32),
            pltpu.VMEM((EB + 16,), jnp.int32),
            pltpu.VMEM((EB + 16,), jnp.int32),
            pltpu.VMEM((G, 256), jnp.float32),
            pltpu.VMEM((ROWS * 256,), jnp.float32),
            pltpu.VMEM((ROWS * 128,), jnp.float32),
            pltpu.SemaphoreType.DMA,
        ],
    )
    return k(ei, xz)


def _prep_body(dega_ref, degb_ref, x_ref, xz_ref, dinv_ref, smean_ref):
    deg = dega_ref[...] + degb_ref[...]
    dinv = lax.rsqrt(deg + 1.0)
    xv = x_ref[...]
    xz_ref[:, :D] = xv
    xz_ref[:, D:] = xv * dinv
    dinv_ref[...] = dinv
    smean_ref[...] = 1.0 / jnp.maximum(deg, 1.0)


def _prep_stage(dega, degb, xpad):
    B = 1024
    grid = (NPAD // B,)
    col1 = pl.BlockSpec((B, 1), lambda i: (i, 0))
    return pl.pallas_call(
        _prep_body,
        grid=grid,
        in_specs=[col1, col1, pl.BlockSpec((B, D), lambda i: (i, 0))],
        out_specs=[pl.BlockSpec((B, 2 * D), lambda i: (i, 0)), col1, col1],
        out_shape=[
            jax.ShapeDtypeStruct((NPAD, 2 * D), jnp.float32),
            jax.ShapeDtypeStruct((NPAD, 1), jnp.float32),
            jax.ShapeDtypeStruct((NPAD, 1), jnp.float32),
        ],
    )(dega, degb, xpad)


def _elu(h):
    return jnp.where(h > 0, h, jnp.exp(jnp.minimum(h, 0.0)) - 1.0)


def _dense_body(wmix_ref, x_ref, s_ref, mx_ref, dinv_ref, smean_ref,
                Wmlp, bmlp, Wsx, Wsm, bs, Wtx, Wtm, bt, Wux, Wum, bu,
                Wg, bg, Wg1, bg1, Wg2, bg2, out_ref):
    x = x_ref[...]
    msum = s_ref[:, :D]
    gsum = s_ref[:, D:]
    dinv = dinv_ref[...]
    mean = msum * smean_ref[...]
    mxv = mx_ref[...]
    mxv = jnp.where(mxv > -1e30, mxv, 0.0)
    agg = gsum * dinv + x * (dinv * dinv)
    f32 = jnp.float32
    h0 = jnp.dot(x, Wmlp[...], preferred_element_type=f32) + bmlp[...]
    h1 = (jnp.dot(x, Wsx[...], preferred_element_type=f32)
          + jnp.dot(mean, Wsm[...], preferred_element_type=f32) + bs[...])
    h2 = (jnp.dot(x, Wtx[...], preferred_element_type=f32)
          + jnp.dot(msum, Wtm[...], preferred_element_type=f32) + bt[...])
    h3 = (jnp.dot(x, Wux[...], preferred_element_type=f32)
          + jnp.dot(mxv, Wum[...], preferred_element_type=f32) + bu[...])
    h4 = jnp.dot(agg, Wg[...], preferred_element_type=f32) + bg[...]
    g = jnp.maximum(jnp.dot(x + msum, Wg1[...], preferred_element_type=f32) + bg1[...], 0.0)
    h5 = jnp.dot(g, Wg2[...], preferred_element_type=f32) + bg2[...]
    w = wmix_ref[...]
    out_ref[...] = (w[0, 0] * _elu(h0) + w[0, 1] * _elu(h1) + w[0, 2] * _elu(h2)
                    + w[0, 3] * _elu(h3) + w[0, 4] * _elu(h4) + w[0, 5] * _elu(h5))


def _dense_stage(wmix, x, S, MX, dinv, smean, *weights):
    B = 1000
    grid = (N // B,)
    row = pl.BlockSpec((B, D), lambda i: (i, 0))
    row2 = pl.BlockSpec((B, 2 * D), lambda i: (i, 0))
    col1 = pl.BlockSpec((B, 1), lambda i: (i, 0))
    full = lambda a: pl.BlockSpec(a.shape, lambda i: (0,) * a.ndim)
    wspecs = [full(w) for w in weights]
    return pl.pallas_call(
        _dense_body,
        grid=grid,
        in_specs=[full(wmix), row, row2, row, col1, col1] + wspecs,
        out_specs=row,
        out_shape=jax.ShapeDtypeStruct((N, D), jnp.float32),
    )(wmix, x, S, MX, dinv, smean, *weights)


def kernel(x, weights, edge_index, W_mlp, b_mlp, W_sage, b_sage, W_ssum, b_ssum,
           W_smax, b_smax, W_gcn, b_gcn, W_gin1, b_gin1, W_gin2, b_gin2):
    src = edge_index[0]
    dst = edge_index[1]

    deg_flat = _deg_stage(dst)
    dega = deg_flat[:NPAD].reshape(NPAD, 1)
    degb = deg_flat[NPAD:].reshape(NPAD, 1)

    xpad = jnp.pad(x, ((0, NPAD - N), (0, 0)))
    xz, dinv, smean = _prep_stage(dega, degb, xpad)

    ei3 = edge_index.reshape(2, NBLK, EB).transpose(1, 0, 2)
    sums_flat, mx_flat = _seg_stage(ei3, xz)
    S = sums_flat.reshape(NPAD, 2 * D)
    MX = mx_flat.reshape(NPAD, D)

    wmix = jnp.zeros((1, 128), jnp.float32).at[0, :6].set(weights)
    b = lambda v: v.reshape(1, D)
    return _dense_stage(wmix, x, S, MX, dinv, smean,
                        W_mlp, b(b_mlp),
                        W_sage[:D], W_sage[D:], b(b_sage),
                        W_ssum[:D], W_ssum[D:], b(b_ssum),
                        W_smax[:D], W_smax[D:], b(b_smax),
                        W_gcn, b(b_gcn),
                        W_gin1, b(b_gin1),
                        W_gin2, b(b_gin2))
